# initial kernel scaffold (unmeasured)
import jax
import jax.numpy as jnp
from jax import lax
from jax.experimental import pallas as pl
from jax.experimental.pallas import tpu as pltpu

M = 8192
HALF = M // 2
D = 4096
BLK = 512
N_BLK = HALF // BLK


def kernel(partial, gamma):
    x2d = partial.reshape(M, D)
    g2d = gamma.reshape(1, D)

    def body(x_ref, g_ref, out_ref, recv_ref, a_ref, b_ref, o_ref,
             send_sem, recv_sem, cp_sems):
        my_x = lax.axis_index("x")
        my_y = lax.axis_index("y")
        my_z = lax.axis_index("z")
        nbr = (my_x, my_y, 1 - my_z)

        barrier = pltpu.get_barrier_semaphore()
        pl.semaphore_signal(barrier, inc=1, device_id=nbr,
                            device_id_type=pl.DeviceIdType.MESH)
        pl.semaphore_wait(barrier, 1)

        theirs = (1 - my_z) * HALF
        mine = my_z * HALF

        rdma = pltpu.make_async_remote_copy(
            src_ref=x_ref.at[pl.ds(theirs, HALF), :],
            dst_ref=recv_ref,
            send_sem=send_sem,
            recv_sem=recv_sem,
            device_id=nbr,
            device_id_type=pl.DeviceIdType.MESH,
        )
        rdma.start()
        rdma.wait()

        def blk(i, carry):
            cp_a = pltpu.make_async_copy(
                x_ref.at[pl.ds(mine + i * BLK, BLK), :], a_ref, cp_sems.at[0])
            cp_b = pltpu.make_async_copy(
                recv_ref.at[pl.ds(i * BLK, BLK), :], b_ref, cp_sems.at[1])
            cp_a.start()
            cp_b.start()
            cp_a.wait()
            cp_b.wait()
            y = a_ref[...] + b_ref[...]
            rms = jnp.sqrt(jnp.mean(y * y, axis=-1, keepdims=True) + 1e-6)
            o_ref[...] = y / rms * g_ref[...]
            cp_o = pltpu.make_async_copy(
                o_ref, out_ref.at[pl.ds(i * BLK, BLK), :], cp_sems.at[2])
            cp_o.start()
            cp_o.wait()
            return carry

        lax.fori_loop(0, N_BLK, blk, 0)

    out, _ = pl.pallas_call(
        body,
        out_shape=(
            jax.ShapeDtypeStruct((HALF, D), jnp.float32),
            jax.ShapeDtypeStruct((HALF, D), jnp.float32),
        ),
        in_specs=[
            pl.BlockSpec(memory_space=pltpu.ANY),
            pl.BlockSpec(memory_space=pltpu.VMEM),
        ],
        out_specs=(
            pl.BlockSpec(memory_space=pltpu.ANY),
            pl.BlockSpec(memory_space=pltpu.ANY),
        ),
        scratch_shapes=[
            pltpu.VMEM((BLK, D), jnp.float32),
            pltpu.VMEM((BLK, D), jnp.float32),
            pltpu.VMEM((BLK, D), jnp.float32),
            pltpu.SemaphoreType.DMA,
            pltpu.SemaphoreType.DMA,
            pltpu.SemaphoreType.DMA((3,)),
        ],
        compiler_params=pltpu.CompilerParams(collective_id=0),
    )(x2d, g2d)
    return out


# baseline (device time: 871846 ns/iter reference)
import jax
import jax.numpy as jnp
from jax import lax
from jax.experimental import pallas as pl
from jax.experimental.pallas import tpu as pltpu

M = 8192
HALF = M // 2
D = 4096
BLK = 256
N_BLK = HALF // BLK


def kernel(partial, gamma):
    x2d = partial.reshape(M, D)
    g2d = gamma.reshape(1, D)

    def body(x_ref, g_ref, out_ref, recv_ref, a_ref, b_ref, o_ref,
             send_sem, recv_sem, cp_sems):
        my_x = lax.axis_index("x")
        my_y = lax.axis_index("y")
        my_z = lax.axis_index("z")
        nbr = (my_x, my_y, 1 - my_z)

        barrier = pltpu.get_barrier_semaphore()
        pl.semaphore_signal(barrier, inc=1, device_id=nbr,
                            device_id_type=pl.DeviceIdType.MESH)
        pl.semaphore_wait(barrier, 1)

        theirs = (1 - my_z) * HALF
        mine = my_z * HALF

        rdma = pltpu.make_async_remote_copy(
            src_ref=x_ref.at[pl.ds(theirs, HALF), :],
            dst_ref=recv_ref,
            send_sem=send_sem,
            recv_sem=recv_sem,
            device_id=nbr,
            device_id_type=pl.DeviceIdType.MESH,
        )
        rdma.start()
        rdma.wait()

        def blk(i, carry):
            cp_a = pltpu.make_async_copy(
                x_ref.at[pl.ds(mine + i * BLK, BLK), :], a_ref, cp_sems.at[0])
            cp_b = pltpu.make_async_copy(
                recv_ref.at[pl.ds(i * BLK, BLK), :], b_ref, cp_sems.at[1])
            cp_a.start()
            cp_b.start()
            cp_a.wait()
            cp_b.wait()
            y = a_ref[...] + b_ref[...]
            rms = jnp.sqrt(jnp.mean(y * y, axis=-1, keepdims=True) + 1e-6)
            o_ref[...] = y / rms * g_ref[...]
            cp_o = pltpu.make_async_copy(
                o_ref, out_ref.at[pl.ds(i * BLK, BLK), :], cp_sems.at[2])
            cp_o.start()
            cp_o.wait()
            return carry

        lax.fori_loop(0, N_BLK, blk, 0)

    out, _ = pl.pallas_call(
        body,
        out_shape=(
            jax.ShapeDtypeStruct((HALF, D), jnp.float32),
            jax.ShapeDtypeStruct((HALF, D), jnp.float32),
        ),
        in_specs=[
            pl.BlockSpec(memory_space=pl.ANY),
            pl.BlockSpec(memory_space=pltpu.VMEM),
        ],
        out_specs=(
            pl.BlockSpec(memory_space=pl.ANY),
            pl.BlockSpec(memory_space=pl.ANY),
        ),
        scratch_shapes=[
            pltpu.VMEM((BLK, D), jnp.float32),
            pltpu.VMEM((BLK, D), jnp.float32),
            pltpu.VMEM((BLK, D), jnp.float32),
            pltpu.SemaphoreType.DMA,
            pltpu.SemaphoreType.DMA,
            pltpu.SemaphoreType.DMA((3,)),
        ],
        compiler_params=pltpu.CompilerParams(collective_id=0),
    )(x2d, g2d)
    return out


# device time: 417596 ns/iter; 2.0878x vs baseline; 2.0878x over previous
import jax
import jax.numpy as jnp
from jax import lax
from jax.experimental import pallas as pl
from jax.experimental.pallas import tpu as pltpu

M = 8192
HALF = M // 2
D = 4096
BLK = 256
N_BLK = HALF // BLK


def kernel(partial, gamma):
    x2d = partial.reshape(M, D)
    g2d = gamma.reshape(1, D)

    def body(x_ref, g_ref, out_ref, recv_ref, ld_ref, sb_ref, a_ref, b_ref,
             o_ref, send_sems, recv_sems, out_sems, cp_sems):
        my_x = lax.axis_index("x")
        my_y = lax.axis_index("y")
        my_z = lax.axis_index("z")
        nbr = (my_x, my_y, 1 - my_z)

        barrier = pltpu.get_barrier_semaphore()
        pl.semaphore_signal(barrier, inc=1, device_id=nbr,
                            device_id_type=pl.DeviceIdType.MESH)
        pl.semaphore_wait(barrier, 1)

        theirs = (1 - my_z) * HALF
        mine = my_z * HALF

        def prep_send(c):
            slot = c % 2
            cp = pltpu.make_async_copy(
                x_ref.at[pl.ds(theirs + c * BLK, BLK), :], ld_ref,
                cp_sems.at[0])
            cp.start()
            cp.wait()
            sb_ref[slot] = ld_ref[...].astype(jnp.bfloat16)
            op = pltpu.make_async_remote_copy(
                src_ref=sb_ref.at[slot],
                dst_ref=recv_ref.at[pl.ds(c * BLK, BLK), :],
                send_sem=send_sems.at[c],
                recv_sem=recv_sems.at[c],
                device_id=nbr,
                device_id_type=pl.DeviceIdType.MESH,
            )
            op.start()
            return op

        ops = [None] * N_BLK
        out_ops = [None] * N_BLK
        ops[0] = prep_send(0)
        for c in range(N_BLK):
            if c + 1 < N_BLK:
                if c - 1 >= 0:
                    ops[c - 1].wait_send()
                ops[c + 1] = prep_send(c + 1)
            ops[c].wait_recv()
            cp_a = pltpu.make_async_copy(
                x_ref.at[pl.ds(mine + c * BLK, BLK), :], a_ref, cp_sems.at[1])
            cp_b = pltpu.make_async_copy(
                recv_ref.at[pl.ds(c * BLK, BLK), :], b_ref, cp_sems.at[2])
            cp_a.start()
            cp_b.start()
            cp_a.wait()
            cp_b.wait()
            y = a_ref[...] + b_ref[...].astype(jnp.float32)
            rms = jnp.sqrt(jnp.mean(y * y, axis=-1, keepdims=True) + 1e-6)
            oslot = c % 2
            if c >= 2:
                out_ops[c - 2].wait()
            o_ref[oslot] = y / rms * g_ref[...]
            out_ops[c] = pltpu.make_async_copy(
                o_ref.at[oslot], out_ref.at[pl.ds(c * BLK, BLK), :],
                out_sems.at[oslot])
            out_ops[c].start()

        ops[N_BLK - 2].wait_send()
        ops[N_BLK - 1].wait_send()
        out_ops[N_BLK - 2].wait()
        out_ops[N_BLK - 1].wait()

    out, _ = pl.pallas_call(
        body,
        out_shape=(
            jax.ShapeDtypeStruct((HALF, D), jnp.float32),
            jax.ShapeDtypeStruct((HALF, D), jnp.bfloat16),
        ),
        in_specs=[
            pl.BlockSpec(memory_space=pl.ANY),
            pl.BlockSpec(memory_space=pltpu.VMEM),
        ],
        out_specs=(
            pl.BlockSpec(memory_space=pl.ANY),
            pl.BlockSpec(memory_space=pl.ANY),
        ),
        scratch_shapes=[
            pltpu.VMEM((BLK, D), jnp.float32),
            pltpu.VMEM((2, BLK, D), jnp.bfloat16),
            pltpu.VMEM((BLK, D), jnp.float32),
            pltpu.VMEM((BLK, D), jnp.bfloat16),
            pltpu.VMEM((2, BLK, D), jnp.float32),
            pltpu.SemaphoreType.DMA((N_BLK,)),
            pltpu.SemaphoreType.DMA((N_BLK,)),
            pltpu.SemaphoreType.DMA((2,)),
            pltpu.SemaphoreType.DMA((3,)),
        ],
        compiler_params=pltpu.CompilerParams(collective_id=0),
    )(x2d, g2d)
    return out


# device time: 238846 ns/iter; 3.6502x vs baseline; 1.7484x over previous
import jax
import jax.numpy as jnp
from jax import lax
from jax.experimental import pallas as pl
from jax.experimental.pallas import tpu as pltpu

M = 8192
HALF = M // 2
HALF2 = HALF // 2
D = 4096
BLK = 256
N_BLK = HALF2 // BLK


def kernel(partial, gamma):
    x2d = partial.reshape(M, D)
    g2d = gamma.reshape(1, D)

    def body(x_ref, g_ref, out_ref, recv_ref, ld_ref, sb_ref, a_ref, b_ref,
             ob_ref, zsend_sems, zrecv_sems, xsend_sems, xrecv_sems,
             ostore_sems, cp_sems):
        my_x = lax.axis_index("x")
        my_y = lax.axis_index("y")
        my_z = lax.axis_index("z")
        znbr = (my_x, my_y, 1 - my_z)
        xnbr = (1 - my_x, my_y, my_z)

        barrier = pltpu.get_barrier_semaphore()
        for nbr in (znbr, xnbr):
            pl.semaphore_signal(barrier, inc=1, device_id=nbr,
                                device_id_type=pl.DeviceIdType.MESH)
        pl.semaphore_wait(barrier, 2)

        theirs = (1 - my_z) * HALF
        mine = my_z * HALF
        part = my_x * HALF2

        def prep_zsend(c):
            slot = c % 2
            cp = pltpu.make_async_copy(
                x_ref.at[pl.ds(theirs + part + c * BLK, BLK), :], ld_ref,
                cp_sems.at[0])
            cp.start()
            cp.wait()
            sb_ref[slot] = ld_ref[...].astype(jnp.bfloat16)
            op = pltpu.make_async_remote_copy(
                src_ref=sb_ref.at[slot],
                dst_ref=recv_ref.at[pl.ds(c * BLK, BLK), :],
                send_sem=zsend_sems.at[c],
                recv_sem=zrecv_sems.at[c],
                device_id=znbr,
                device_id_type=pl.DeviceIdType.MESH,
            )
            op.start()
            return op

        zops = [None] * N_BLK
        xops = [None] * N_BLK
        oops = [None] * N_BLK
        zops[0] = prep_zsend(0)
        for c in range(N_BLK):
            if c + 1 < N_BLK:
                if c - 1 >= 0:
                    zops[c - 1].wait_send()
                zops[c + 1] = prep_zsend(c + 1)
            zops[c].wait_recv()
            cp_a = pltpu.make_async_copy(
                x_ref.at[pl.ds(mine + part + c * BLK, BLK), :], a_ref,
                cp_sems.at[1])
            cp_b = pltpu.make_async_copy(
                recv_ref.at[pl.ds(c * BLK, BLK), :], b_ref, cp_sems.at[2])
            cp_a.start()
            cp_b.start()
            cp_a.wait()
            cp_b.wait()
            y = a_ref[...] + b_ref[...].astype(jnp.float32)
            rms = jnp.sqrt(jnp.mean(y * y, axis=-1, keepdims=True) + 1e-6)
            oslot = c % 2
            if c >= 2:
                oops[c - 2].wait()
                xops[c - 2].wait_send()
            ob_ref[oslot] = (y / rms * g_ref[...]).astype(jnp.bfloat16)
            oops[c] = pltpu.make_async_copy(
                ob_ref.at[oslot], out_ref.at[pl.ds(part + c * BLK, BLK), :],
                ostore_sems.at[oslot])
            oops[c].start()
            xops[c] = pltpu.make_async_remote_copy(
                src_ref=ob_ref.at[oslot],
                dst_ref=out_ref.at[pl.ds(part + c * BLK, BLK), :],
                send_sem=xsend_sems.at[c],
                recv_sem=xrecv_sems.at[c],
                device_id=xnbr,
                device_id_type=pl.DeviceIdType.MESH,
            )
            xops[c].start()

        zops[N_BLK - 2].wait_send()
        zops[N_BLK - 1].wait_send()
        oops[N_BLK - 2].wait()
        oops[N_BLK - 1].wait()
        xops[N_BLK - 2].wait_send()
        xops[N_BLK - 1].wait_send()
        for c in range(N_BLK):
            xops[c].wait_recv()

    out, _ = pl.pallas_call(
        body,
        out_shape=(
            jax.ShapeDtypeStruct((HALF, D), jnp.bfloat16),
            jax.ShapeDtypeStruct((HALF2, D), jnp.bfloat16),
        ),
        in_specs=[
            pl.BlockSpec(memory_space=pl.ANY),
            pl.BlockSpec(memory_space=pltpu.VMEM),
        ],
        out_specs=(
            pl.BlockSpec(memory_space=pl.ANY),
            pl.BlockSpec(memory_space=pl.ANY),
        ),
        scratch_shapes=[
            pltpu.VMEM((BLK, D), jnp.float32),
            pltpu.VMEM((2, BLK, D), jnp.bfloat16),
            pltpu.VMEM((BLK, D), jnp.float32),
            pltpu.VMEM((BLK, D), jnp.bfloat16),
            pltpu.VMEM((2, BLK, D), jnp.bfloat16),
            pltpu.SemaphoreType.DMA((N_BLK,)),
            pltpu.SemaphoreType.DMA((N_BLK,)),
            pltpu.SemaphoreType.DMA((N_BLK,)),
            pltpu.SemaphoreType.DMA((N_BLK,)),
            pltpu.SemaphoreType.DMA((2,)),
            pltpu.SemaphoreType.DMA((3,)),
        ],
        compiler_params=pltpu.CompilerParams(collective_id=0),
    )(x2d, g2d)
    return out


# device time: 191727 ns/iter; 4.5473x vs baseline; 1.2458x over previous
import jax
import jax.numpy as jnp
from jax import lax
from jax.experimental import pallas as pl
from jax.experimental.pallas import tpu as pltpu

M = 8192
HALF = M // 2
HALF2 = HALF // 2
D = 4096
BLK = 256
N_CH = HALF2 // BLK
N_XD = 5
N_SW = 3


def kernel(partial, gamma):
    x2d = partial.reshape(M, D)
    g2d = gamma.reshape(1, D)

    def body(x_ref, g_ref, out_ref, contrib_ref, ld_ref, sb_ref, a_ref,
             b_ref, ob_ref, zsend, zrecv, swsend, swrecv, xsend, xrecv,
             fwsend, fwrecv, ostore, cp_sems):
        my_x = lax.axis_index("x")
        my_y = lax.axis_index("y")
        my_z = lax.axis_index("z")
        znbr = (my_x, my_y, 1 - my_z)
        xnbr = (1 - my_x, my_y, my_z)
        ynbr = (my_x, 1 - my_y, my_z)

        barrier = pltpu.get_barrier_semaphore()
        for nbr in (znbr, xnbr, ynbr):
            pl.semaphore_signal(barrier, inc=1, device_id=nbr,
                                device_id_type=pl.DeviceIdType.MESH)
        pl.semaphore_wait(barrier, 3)

        theirs = (1 - my_z) * HALF
        mine = my_z * HALF
        part = my_x * HALF2
        partner = (1 - my_x) * HALF2

        def a_of(k):
            return my_y * 5 + k if k < N_SW else k

        def s_of(k):
            return (1 - my_y) * 5 + k

        def prep_zsend(k):
            slot = k % 2
            ac = a_of(k)
            cp = pltpu.make_async_copy(
                x_ref.at[pl.ds(theirs + part + ac * BLK, BLK), :], ld_ref,
                cp_sems.at[0])
            cp.start()
            cp.wait()
            sb_ref[slot] = ld_ref[...].astype(jnp.bfloat16)
            op = pltpu.make_async_remote_copy(
                src_ref=sb_ref.at[slot],
                dst_ref=contrib_ref.at[pl.ds(ac * BLK, BLK), :],
                send_sem=zsend.at[k],
                recv_sem=zrecv.at[k],
                device_id=znbr,
                device_id_type=pl.DeviceIdType.MESH,
            )
            op.start()
            return op

        users = {}

        def compute_chunk(ac, c, k_x):
            cp_a = pltpu.make_async_copy(
                x_ref.at[pl.ds(mine + part + ac * BLK, BLK), :], a_ref,
                cp_sems.at[1])
            cp_b = pltpu.make_async_copy(
                contrib_ref.at[pl.ds(ac * BLK, BLK), :], b_ref,
                cp_sems.at[2])
            cp_a.start()
            cp_b.start()
            cp_a.wait()
            cp_b.wait()
            y = a_ref[...] + b_ref[...].astype(jnp.float32)
            rms = jnp.sqrt(jnp.mean(y * y, axis=-1, keepdims=True) + 1e-6)
            oslot = c % 2
            if c >= 2:
                for kind, op in users[c - 2]:
                    op.wait() if kind == "l" else op.wait_send()
            ob_ref[oslot] = (y / rms * g_ref[...]).astype(jnp.bfloat16)
            o_op = pltpu.make_async_copy(
                ob_ref.at[oslot], out_ref.at[pl.ds(part + ac * BLK, BLK), :],
                ostore.at[oslot])
            o_op.start()
            users[c] = [("l", o_op)]
            if k_x is not None:
                x_op = pltpu.make_async_remote_copy(
                    src_ref=ob_ref.at[oslot],
                    dst_ref=out_ref.at[pl.ds(part + ac * BLK, BLK), :],
                    send_sem=xsend.at[k_x],
                    recv_sem=xrecv.at[k_x],
                    device_id=xnbr,
                    device_id_type=pl.DeviceIdType.MESH,
                )
                x_op.start()
                users[c].append(("s", x_op))
                return x_op
            return None

        zops = {0: prep_zsend(0)}
        xops = {}
        swops = {}
        fwops = {}
        for k in range(N_XD):
            if k + 1 < N_XD:
                if k - 1 >= 0:
                    zops[k - 1].wait_send()
                zops[k + 1] = prep_zsend(k + 1)
            zops[k].wait_recv()
            if k < N_SW:
                swops[k] = pltpu.make_async_remote_copy(
                    src_ref=contrib_ref.at[pl.ds(a_of(k) * BLK, BLK), :],
                    dst_ref=contrib_ref.at[pl.ds(a_of(k) * BLK, BLK), :],
                    send_sem=swsend.at[k],
                    recv_sem=swrecv.at[k],
                    device_id=ynbr,
                    device_id_type=pl.DeviceIdType.MESH,
                )
                swops[k].start()
            xops[k] = compute_chunk(a_of(k), k, k)
            if k >= 1:
                xops[k - 1].wait_recv()
                if k - 1 < N_SW:
                    fwops[k - 1] = pltpu.make_async_remote_copy(
                        src_ref=out_ref.at[
                            pl.ds(partner + a_of(k - 1) * BLK, BLK), :],
                        dst_ref=out_ref.at[
                            pl.ds(partner + a_of(k - 1) * BLK, BLK), :],
                        send_sem=fwsend.at[k - 1],
                        recv_sem=fwrecv.at[k - 1],
                        device_id=ynbr,
                        device_id_type=pl.DeviceIdType.MESH,
                    )
                    fwops[k - 1].start()
        xops[N_XD - 1].wait_recv()

        for k in range(N_SW):
            swops[k].wait_recv()
            compute_chunk(s_of(k), N_XD + k, None)

        zops[N_XD - 2].wait_send()
        zops[N_XD - 1].wait_send()
        for k in range(N_SW):
            swops[k].wait_send()
            fwops[k].wait_send()
        for k in range(N_SW):
            fwops[k].wait_recv()
        for c in (N_CH - 2, N_CH - 1):
            for kind, op in users[c]:
                op.wait() if kind == "l" else op.wait_send()

    out, _ = pl.pallas_call(
        body,
        out_shape=(
            jax.ShapeDtypeStruct((HALF, D), jnp.bfloat16),
            jax.ShapeDtypeStruct((HALF2, D), jnp.bfloat16),
        ),
        in_specs=[
            pl.BlockSpec(memory_space=pl.ANY),
            pl.BlockSpec(memory_space=pltpu.VMEM),
        ],
        out_specs=(
            pl.BlockSpec(memory_space=pl.ANY),
            pl.BlockSpec(memory_space=pl.ANY),
        ),
        scratch_shapes=[
            pltpu.VMEM((BLK, D), jnp.float32),
            pltpu.VMEM((2, BLK, D), jnp.bfloat16),
            pltpu.VMEM((BLK, D), jnp.float32),
            pltpu.VMEM((BLK, D), jnp.bfloat16),
            pltpu.VMEM((2, BLK, D), jnp.bfloat16),
            pltpu.SemaphoreType.DMA((N_XD,)),
            pltpu.SemaphoreType.DMA((N_XD,)),
            pltpu.SemaphoreType.DMA((N_SW,)),
            pltpu.SemaphoreType.DMA((N_SW,)),
            pltpu.SemaphoreType.DMA((N_XD,)),
            pltpu.SemaphoreType.DMA((N_XD,)),
            pltpu.SemaphoreType.DMA((N_SW,)),
            pltpu.SemaphoreType.DMA((N_SW,)),
            pltpu.SemaphoreType.DMA((2,)),
            pltpu.SemaphoreType.DMA((3,)),
        ],
        compiler_params=pltpu.CompilerParams(collective_id=0),
    )(x2d, g2d)
    return out
